# baseline (speedup 1.0000x reference)
import functools

import jax
import jax.numpy as jnp
from jax import lax
from jax.experimental import pallas as pl
from jax.experimental.pallas import tpu as pltpu
from jax.experimental.pallas import tpu_sc as plsc

N = 100000
C = 32
K = 27

N_PAD = 102400
M_PAD = N_PAD * K
GWIN = 256

TC_BLK = 512


def _sc_gather(table, idx):
  mesh = plsc.VectorSubcoreMesh(core_axis_name="c", subcore_axis_name="s")

  @functools.partial(
      pl.kernel,
      mesh=mesh,
      compiler_params=pltpu.CompilerParams(use_tc_tiling_on_sc=False),
      out_type=jax.ShapeDtypeStruct((M_PAD, C), table.dtype),
  )
  def k(table_hbm, idx_hbm, out_hbm):
    def body(i_vmem, o_vmem):
      pltpu.sync_copy(table_hbm.at[i_vmem.at[0]], o_vmem)

    pltpu.emit_pipeline(
        body,
        grid=(M_PAD // GWIN,),
        in_specs=[pl.BlockSpec((1, GWIN), index_map=lambda i: (0, i))],
        out_specs=[pl.BlockSpec((GWIN, C), index_map=lambda i: (i, 0))],
        core_axis_name=("c", "s"),
        dimension_semantics=(pltpu.PARALLEL,),
    )(idx_hbm, out_hbm)

  return k(table, idx)


def _tc_stage(g, wcat, gamma, beta, res, zero_tail, out_dtype):
  nblk = N_PAD // TC_BLK

  def body(g_ref, w_ref, ga_ref, be_ref, *rest):
    if res is not None:
      r_ref, o_ref = rest
    else:
      (o_ref,) = rest
    acc = jnp.dot(g_ref[...], w_ref[...], preferred_element_type=jnp.float32)
    m = jnp.mean(acc, axis=1, keepdims=True)
    v = jnp.mean((acc - m) ** 2, axis=1, keepdims=True)
    y = (acc - m) * lax.rsqrt(v + 1e-5) * ga_ref[...] + be_ref[...]
    if res is not None:
      y = y + r_ref[...]
    y = jnp.maximum(y, 0.0)
    if zero_tail:
      i = pl.program_id(0)
      row = i * TC_BLK + lax.broadcasted_iota(jnp.int32, (TC_BLK, 1), 0)
      y = jnp.where(row < N, y, 0.0)
    o_ref[...] = y.astype(out_dtype)

  in_specs = [
      pl.BlockSpec((TC_BLK, K * C), lambda i: (i, 0)),
      pl.BlockSpec((K * C, C), lambda i: (0, 0)),
      pl.BlockSpec((1, C), lambda i: (0, 0)),
      pl.BlockSpec((1, C), lambda i: (0, 0)),
  ]
  args = [g, wcat, gamma.reshape(1, C), beta.reshape(1, C)]
  if res is not None:
    in_specs.append(pl.BlockSpec((TC_BLK, C), lambda i: (i, 0)))
    args.append(res)

  return pl.pallas_call(
      body,
      grid=(nblk,),
      in_specs=in_specs,
      out_specs=pl.BlockSpec((TC_BLK, C), lambda i: (i, 0)),
      out_shape=jax.ShapeDtypeStruct((N_PAD, C), out_dtype),
  )(*args)


def kernel(features, W1, g1, b1, W2, g2, b2, nbr):
  f_pad = jnp.pad(features, ((0, N_PAD - N), (0, 0)))
  idx = jnp.pad(nbr.reshape(-1), (0, M_PAD - N * K), constant_values=-1)
  z = N + (jnp.arange(M_PAD, dtype=jnp.int32) & 2047)
  idx = jnp.where(idx < 0, z, idx).reshape(1, M_PAD)
  g1rows = _sc_gather(f_pad, idx)
  out1 = _tc_stage(g1rows.reshape(N_PAD, K * C), W1.reshape(K * C, C),
                   g1, b1, None, zero_tail=True, out_dtype=jnp.float32)
  g2rows = _sc_gather(out1, idx)
  out = _tc_stage(g2rows.reshape(N_PAD, K * C), W2.reshape(K * C, C),
                  g2, b2, f_pad, zero_tail=False, out_dtype=jnp.float32)
  return out[:N]


# speedup vs baseline: 2.1982x; 2.1982x over previous
import functools

import jax
import jax.numpy as jnp
from jax import lax
from jax.experimental import pallas as pl
from jax.experimental.pallas import tpu as pltpu
from jax.experimental.pallas import tpu_sc as plsc

N = 100000
C = 32
K = 27

N_PAD = 102400
M_PAD = N_PAD * K
GWIN = 64

TC_BLK = 512


def _sc_gather(table, idx):
  mesh = plsc.VectorSubcoreMesh(core_axis_name="c", subcore_axis_name="s")

  @functools.partial(
      pl.kernel,
      mesh=mesh,
      compiler_params=pltpu.CompilerParams(use_tc_tiling_on_sc=False),
      out_type=jax.ShapeDtypeStruct((M_PAD, C), table.dtype),
  )
  def k(table_hbm, idx_hbm, out_hbm):
    def body(i_vmem, o_vmem):
      pltpu.sync_copy(table_hbm.at[i_vmem.at[0]], o_vmem)

    pltpu.emit_pipeline(
        body,
        grid=(M_PAD // GWIN,),
        in_specs=[pl.BlockSpec((1, GWIN), index_map=lambda i: (0, i))],
        out_specs=[pl.BlockSpec((GWIN, C), index_map=lambda i: (i, 0))],
        core_axis_name=("c", "s"),
        dimension_semantics=(pltpu.PARALLEL,),
    )(idx_hbm, out_hbm)

  return k(table, idx)


def _tc_stage(g, wcat, gamma, beta, res, zero_tail, out_dtype):
  nblk = N_PAD // TC_BLK

  def body(g_ref, w_ref, ga_ref, be_ref, *rest):
    if res is not None:
      r_ref, o_ref = rest
    else:
      (o_ref,) = rest
    acc = jnp.dot(g_ref[...], w_ref[...], preferred_element_type=jnp.float32)
    m = jnp.mean(acc, axis=1, keepdims=True)
    v = jnp.mean((acc - m) ** 2, axis=1, keepdims=True)
    y = (acc - m) * lax.rsqrt(v + 1e-5) * ga_ref[...] + be_ref[...]
    if res is not None:
      y = y + r_ref[...]
    y = jnp.maximum(y, 0.0)
    if zero_tail:
      i = pl.program_id(0)
      row = i * TC_BLK + lax.broadcasted_iota(jnp.int32, (TC_BLK, 1), 0)
      y = jnp.where(row < N, y, 0.0)
    o_ref[...] = y.astype(out_dtype)

  in_specs = [
      pl.BlockSpec((TC_BLK, K * C), lambda i: (i, 0)),
      pl.BlockSpec((K * C, C), lambda i: (0, 0)),
      pl.BlockSpec((1, C), lambda i: (0, 0)),
      pl.BlockSpec((1, C), lambda i: (0, 0)),
  ]
  args = [g, wcat, gamma.reshape(1, C), beta.reshape(1, C)]
  if res is not None:
    in_specs.append(pl.BlockSpec((TC_BLK, C), lambda i: (i, 0)))
    args.append(res)

  return pl.pallas_call(
      body,
      grid=(nblk,),
      in_specs=in_specs,
      out_specs=pl.BlockSpec((TC_BLK, C), lambda i: (i, 0)),
      out_shape=jax.ShapeDtypeStruct((N_PAD, C), out_dtype),
  )(*args)


def kernel(features, W1, g1, b1, W2, g2, b2, nbr):
  f_pad = jnp.pad(features, ((0, N_PAD - N), (0, 0)))
  idx = jnp.pad(nbr.reshape(-1), (0, M_PAD - N * K), constant_values=-1)
  z = N + (jnp.arange(M_PAD, dtype=jnp.int32) & 2047)
  idx = jnp.where(idx < 0, z, idx).reshape(1, M_PAD)
  g1rows = _sc_gather(f_pad, idx)
  out1 = _tc_stage(g1rows.reshape(N_PAD, K * C), W1.reshape(K * C, C),
                   g1, b1, None, zero_tail=True, out_dtype=jnp.float32)
  g2rows = _sc_gather(out1, idx)
  out = _tc_stage(g2rows.reshape(N_PAD, K * C), W2.reshape(K * C, C),
                  g2, b2, f_pad, zero_tail=False, out_dtype=jnp.float32)
  return out[:N]


# speedup vs baseline: 2.3733x; 1.0797x over previous
import functools

import jax
import jax.numpy as jnp
from jax import lax
from jax.experimental import pallas as pl
from jax.experimental.pallas import tpu as pltpu
from jax.experimental.pallas import tpu_sc as plsc

N = 100000
C = 32
K = 27

N_PAD = 102400
M_PAD = N_PAD * K
GWIN = 32

TC_BLK = 512


def _sc_gather(table, idx):
  mesh = plsc.VectorSubcoreMesh(core_axis_name="c", subcore_axis_name="s")

  @functools.partial(
      pl.kernel,
      mesh=mesh,
      compiler_params=pltpu.CompilerParams(use_tc_tiling_on_sc=False),
      out_type=jax.ShapeDtypeStruct((M_PAD, C), table.dtype),
  )
  def k(table_hbm, idx_hbm, out_hbm):
    def body(i_vmem, o_vmem):
      pltpu.sync_copy(table_hbm.at[i_vmem.at[0]], o_vmem)

    pltpu.emit_pipeline(
        body,
        grid=(M_PAD // GWIN,),
        in_specs=[pl.BlockSpec((1, GWIN), index_map=lambda i: (0, i))],
        out_specs=[pl.BlockSpec((GWIN, C), index_map=lambda i: (i, 0))],
        core_axis_name=("c", "s"),
        dimension_semantics=(pltpu.PARALLEL,),
    )(idx_hbm, out_hbm)

  return k(table, idx)


def _tc_stage(g, wcat, gamma, beta, res, zero_tail, out_dtype):
  nblk = N_PAD // TC_BLK

  def body(g_ref, w_ref, ga_ref, be_ref, *rest):
    if res is not None:
      r_ref, o_ref = rest
    else:
      (o_ref,) = rest
    acc = jnp.dot(g_ref[...], w_ref[...], preferred_element_type=jnp.float32)
    m = jnp.mean(acc, axis=1, keepdims=True)
    v = jnp.mean((acc - m) ** 2, axis=1, keepdims=True)
    y = (acc - m) * lax.rsqrt(v + 1e-5) * ga_ref[...] + be_ref[...]
    if res is not None:
      y = y + r_ref[...]
    y = jnp.maximum(y, 0.0)
    if zero_tail:
      i = pl.program_id(0)
      row = i * TC_BLK + lax.broadcasted_iota(jnp.int32, (TC_BLK, 1), 0)
      y = jnp.where(row < N, y, 0.0)
    o_ref[...] = y.astype(out_dtype)

  in_specs = [
      pl.BlockSpec((TC_BLK, K * C), lambda i: (i, 0)),
      pl.BlockSpec((K * C, C), lambda i: (0, 0)),
      pl.BlockSpec((1, C), lambda i: (0, 0)),
      pl.BlockSpec((1, C), lambda i: (0, 0)),
  ]
  args = [g, wcat, gamma.reshape(1, C), beta.reshape(1, C)]
  if res is not None:
    in_specs.append(pl.BlockSpec((TC_BLK, C), lambda i: (i, 0)))
    args.append(res)

  return pl.pallas_call(
      body,
      grid=(nblk,),
      in_specs=in_specs,
      out_specs=pl.BlockSpec((TC_BLK, C), lambda i: (i, 0)),
      out_shape=jax.ShapeDtypeStruct((N_PAD, C), out_dtype),
  )(*args)


def kernel(features, W1, g1, b1, W2, g2, b2, nbr):
  f_pad = jnp.pad(features, ((0, N_PAD - N), (0, 0)))
  idx = jnp.pad(nbr.reshape(-1), (0, M_PAD - N * K), constant_values=-1)
  z = N + (jnp.arange(M_PAD, dtype=jnp.int32) & 2047)
  idx = jnp.where(idx < 0, z, idx).reshape(1, M_PAD)
  g1rows = _sc_gather(f_pad, idx)
  out1 = _tc_stage(g1rows.reshape(N_PAD, K * C), W1.reshape(K * C, C),
                   g1, b1, None, zero_tail=True, out_dtype=jnp.float32)
  g2rows = _sc_gather(out1, idx)
  out = _tc_stage(g2rows.reshape(N_PAD, K * C), W2.reshape(K * C, C),
                  g2, b2, f_pad, zero_tail=False, out_dtype=jnp.float32)
  return out[:N]
